# baseline (speedup 1.0000x reference)
import functools
import math

import jax
import jax.numpy as jnp
from jax import lax
from jax.experimental import pallas as pl
from jax.experimental.pallas import tpu as pltpu
from jax.experimental.pallas import tpu_sc as plsc

D = 64
LANES = 16
NW = 32
CHUNK = 128
NBUF = 8
AHEAD = 4


def _pe_table(n_rows: int, dim: int) -> jax.Array:
    position = jnp.arange(n_rows, dtype=jnp.float32)[:, None]
    div_term = jnp.exp(
        jnp.arange(0.0, dim, 2, dtype=jnp.float32) * -(math.log(10000.0) / dim)
    )
    tmp = position * div_term
    pe = jnp.zeros((n_rows, dim), dtype=jnp.float32)
    pe = pe.at[:, 0::2].set(jnp.sin(tmp))
    pe = pe.at[:, 1::2].set(jnp.cos(tmp))
    return pe


@functools.partial(jax.jit, static_argnums=(2, 3))
def _embed_sc(idx3, table, batch, seq_len):
    n_tokens = batch * seq_len
    tokens_per_w = n_tokens // NW
    n_chunks = tokens_per_w // CHUNK
    n_rounds = n_chunks // NBUF
    assert tokens_per_w % CHUNK == 0 and n_chunks % NBUF == 0
    assert tokens_per_w % seq_len == 0

    nbuf = NBUF
    ahead = AHEAD
    scale = math.sqrt(D)

    pe_rows = seq_len - math.gcd(CHUNK, seq_len) + CHUNK
    pe_base = _pe_table(seq_len, D)
    pe_ext = jnp.concatenate([pe_base, pe_base[: pe_rows - seq_len]], axis=0)

    mesh = plsc.VectorSubcoreMesh(
        core_axis_name="c", subcore_axis_name="s", num_cores=2, num_subcores=16
    )

    @functools.partial(
        pl.kernel,
        out_type=jax.ShapeDtypeStruct((n_tokens, D), jnp.float32),
        mesh=mesh,
        scratch_types=[
            pltpu.VMEM((n_chunks, CHUNK), jnp.int32),
            pltpu.VMEM((pe_rows, D), jnp.float32),
            pltpu.VMEM((nbuf, CHUNK, D), jnp.float32),
            [pltpu.SemaphoreType.DMA] * nbuf,
            [pltpu.SemaphoreType.DMA] * nbuf,
        ],
        compiler_params=pltpu.CompilerParams(use_tc_tiling_on_sc=False),
    )
    def k(idx_hbm, table_hbm, pe_hbm, out_hbm, idx_v, pe_v, rows_v, sem_g, sem_o):
        wid = lax.axis_index("s") * 2 + lax.axis_index("c")
        base = wid * tokens_per_w
        pltpu.sync_copy(idx_hbm.at[wid], idx_v)
        pltpu.sync_copy(pe_hbm, pe_v)

        def g_start(g, b):
            pltpu.async_copy(table_hbm.at[idx_v.at[g]], rows_v.at[b], sem_g[b])

        def g_wait(g, b):
            pltpu.make_async_copy(
                table_hbm.at[idx_v.at[g]], rows_v.at[b], sem_g[b]
            ).wait()

        def o_start(g, b):
            pltpu.async_copy(
                rows_v.at[b], out_hbm.at[pl.ds(base + g * CHUNK, CHUNK)], sem_o[b]
            )

        def o_wait(g, b):
            pltpu.make_async_copy(
                rows_v.at[b], out_hbm.at[pl.ds(base + g * CHUNK, CHUNK)], sem_o[b]
            ).wait()

        for g0 in range(ahead):
            g_start(g0, g0)

        def round_body(r, carry):
            for b in range(nbuf):
                g = r * nbuf + b
                bt = (b + ahead) % nbuf
                gp = g + ahead
                gv = gp - nbuf

                @pl.when(gp < n_chunks)
                def _():
                    @pl.when(gv >= 0)
                    def _():
                        o_wait(gv, bt)

                    g_start(gp, bt)

                g_wait(g, b)
                p0 = lax.rem(g * CHUNK, seq_len)

                def t_body(t, c):
                    for j in range(D // LANES):
                        s = pl.ds(j * LANES, LANES)
                        rows_v[b, t, s] = rows_v[b, t, s] * scale + pe_v[p0 + t, s]
                    return c

                lax.fori_loop(0, CHUNK, t_body, 0, unroll=2)
                o_start(g, b)
            return carry

        lax.fori_loop(0, n_rounds, round_body, 0, unroll=False)

        for b in range(nbuf):
            o_wait(n_chunks - nbuf + b, b)

    return k(idx3, table, pe_ext)


def kernel(inputs, embed_weight):
    batch, seq_len = inputs.shape
    tokens_per_w = batch * seq_len // NW
    idx3 = inputs.reshape(NW, tokens_per_w // CHUNK, CHUNK)
    out = _embed_sc(idx3, embed_weight, batch, seq_len)
    return out.reshape(batch, seq_len, D)


# speedup vs baseline: 1.1945x; 1.1945x over previous
import functools
import math

import jax
import jax.numpy as jnp
from jax import lax
from jax.experimental import pallas as pl
from jax.experimental.pallas import tpu as pltpu
from jax.experimental.pallas import tpu_sc as plsc

D = 64
LANES = 16
NW = 32
CH0 = 104
NBUF = 8
AHEAD = 4


def _pe_table(n_rows: int, dim: int) -> jax.Array:
    position = jnp.arange(n_rows, dtype=jnp.float32)[:, None]
    div_term = jnp.exp(
        jnp.arange(0.0, dim, 2, dtype=jnp.float32) * -(math.log(10000.0) / dim)
    )
    tmp = position * div_term
    pe = jnp.zeros((n_rows, dim), dtype=jnp.float32)
    pe = pe.at[:, 0::2].set(jnp.sin(tmp))
    pe = pe.at[:, 1::2].set(jnp.cos(tmp))
    return pe


@functools.partial(jax.jit, static_argnums=(2, 3))
def _embed_sc(inputs, table, batch, seq_len):
    seqs_per_w = batch // NW
    ch1 = seq_len - CH0
    n_chunks = seqs_per_w * 2
    n_rounds = n_chunks // NBUF
    assert 0 < ch1 <= 128 and CH0 % 8 == 0 and ch1 % 8 == 0
    assert batch % NW == 0 and n_chunks % NBUF == 0 and NBUF % 2 == 0

    nbuf = NBUF
    ahead = AHEAD
    scale = math.sqrt(D)
    pe = _pe_table(seq_len, D)

    mesh = plsc.VectorSubcoreMesh(
        core_axis_name="c", subcore_axis_name="s", num_cores=2, num_subcores=16
    )

    @functools.partial(
        pl.kernel,
        out_type=jax.ShapeDtypeStruct((batch, seq_len, D), jnp.float32),
        mesh=mesh,
        scratch_types=[
            pltpu.VMEM((seqs_per_w, seq_len), jnp.int32),
            pltpu.VMEM((seq_len, D), jnp.float32),
            pltpu.VMEM((nbuf, CH0, D), jnp.float32),
            [pltpu.SemaphoreType.DMA] * nbuf,
            [pltpu.SemaphoreType.DMA] * nbuf,
        ],
        compiler_params=pltpu.CompilerParams(use_tc_tiling_on_sc=False),
    )
    def k(idx_hbm, table_hbm, pe_hbm, out_hbm, idx_v, pe_v, rows_v, sem_g, sem_o):
        wid = lax.axis_index("s") * 2 + lax.axis_index("c")
        seq0 = wid * seqs_per_w
        pltpu.sync_copy(idx_hbm.at[pl.ds(seq0, seqs_per_w)], idx_v)
        pltpu.sync_copy(pe_hbm, pe_v)

        def chunk_refs(g, parity, b):
            s_loc = lax.div(g, 2)
            off = parity * CH0
            size = ch1 if parity else CH0
            src = table_hbm.at[idx_v.at[s_loc, pl.ds(off, size)]]
            dst = out_hbm.at[seq0 + s_loc, pl.ds(off, size)]
            return src, dst, size

        def g_start(g, parity, b):
            src, _, size = chunk_refs(g, parity, b)
            pltpu.async_copy(src, rows_v.at[b, pl.ds(0, size)], sem_g[b])

        def g_wait(g, parity, b):
            src, _, size = chunk_refs(g, parity, b)
            pltpu.make_async_copy(src, rows_v.at[b, pl.ds(0, size)], sem_g[b]).wait()

        def o_start(g, parity, b):
            _, dst, size = chunk_refs(g, parity, b)
            pltpu.async_copy(rows_v.at[b, pl.ds(0, size)], dst, sem_o[b])

        def o_wait(g, parity, b):
            _, dst, size = chunk_refs(g, parity, b)
            pltpu.make_async_copy(rows_v.at[b, pl.ds(0, size)], dst, sem_o[b]).wait()

        for g0 in range(ahead):
            g_start(g0, g0 % 2, g0)

        def round_body(r, carry):
            for b in range(nbuf):
                g = r * nbuf + b
                parity = b % 2
                bt = (b + ahead) % nbuf
                gp = g + ahead
                gv = gp - nbuf

                @pl.when(gp < n_chunks)
                def _():
                    @pl.when(gv >= 0)
                    def _():
                        o_wait(gv, bt % 2, bt)

                    g_start(gp, bt % 2, bt)

                g_wait(g, parity, b)
                p0 = parity * CH0
                size = ch1 if parity else CH0

                def t_body(t, c):
                    for j in range(D // LANES):
                        s = pl.ds(j * LANES, LANES)
                        rows_v[b, t, s] = rows_v[b, t, s] * scale + pe_v[p0 + t, s]
                    return c

                lax.fori_loop(0, size, t_body, 0, unroll=2)
                o_start(g, parity, b)
            return carry

        lax.fori_loop(0, n_rounds, round_body, 0, unroll=False)

        for b in range(nbuf):
            o_wait(n_chunks - nbuf + b, b % 2, b)

    return k(inputs, table, pe)


def kernel(inputs, embed_weight):
    batch, seq_len = inputs.shape
    return _embed_sc(inputs, embed_weight, batch, seq_len)


# speedup vs baseline: 1.3092x; 1.0960x over previous
import functools
import math

import jax
import jax.numpy as jnp
from jax import lax
from jax.experimental import pallas as pl
from jax.experimental.pallas import tpu as pltpu
from jax.experimental.pallas import tpu_sc as plsc

D = 64
LANES = 16
NW = 32
NBUF = 8
AHEAD = 4


def _pe_table(n_rows: int, dim: int) -> jax.Array:
    position = jnp.arange(n_rows, dtype=jnp.float32)[:, None]
    div_term = jnp.exp(
        jnp.arange(0.0, dim, 2, dtype=jnp.float32) * -(math.log(10000.0) / dim)
    )
    tmp = position * div_term
    pe = jnp.zeros((n_rows, dim), dtype=jnp.float32)
    pe = pe.at[:, 0::2].set(jnp.sin(tmp))
    pe = pe.at[:, 1::2].set(jnp.cos(tmp))
    return pe


@functools.partial(jax.jit, static_argnums=(2, 3))
def _embed_sc(inputs_t, table, batch, seq_len):
    seqs_per_w = batch // NW
    n_chunks = seq_len
    assert batch % NW == 0 and seqs_per_w % 8 == 0
    assert n_chunks % NBUF == 0 and seqs_per_w <= 128
    n_rounds = n_chunks // NBUF

    nbuf = NBUF
    ahead = AHEAD
    scale = math.sqrt(D)
    pe = _pe_table(seq_len, D)

    mesh = plsc.VectorSubcoreMesh(
        core_axis_name="c", subcore_axis_name="s", num_cores=2, num_subcores=16
    )

    @functools.partial(
        pl.kernel,
        out_type=jax.ShapeDtypeStruct((batch, seq_len, D), jnp.float32),
        mesh=mesh,
        scratch_types=[
            pltpu.VMEM((seq_len, seqs_per_w), jnp.int32),
            pltpu.VMEM((seq_len, D), jnp.float32),
            pltpu.VMEM((nbuf, seqs_per_w, D), jnp.float32),
            [pltpu.SemaphoreType.DMA] * nbuf,
            [pltpu.SemaphoreType.DMA] * nbuf,
        ],
        compiler_params=pltpu.CompilerParams(use_tc_tiling_on_sc=False),
    )
    def k(idx_hbm, table_hbm, pe_hbm, out_hbm, idx_v, pe_v, rows_v, sem_g, sem_o):
        wid = lax.axis_index("s") * 2 + lax.axis_index("c")
        seq0 = wid * seqs_per_w
        pltpu.sync_copy(idx_hbm.at[:, pl.ds(seq0, seqs_per_w)], idx_v)
        pltpu.sync_copy(pe_hbm, pe_v)

        def g_start(g, b):
            pltpu.async_copy(table_hbm.at[idx_v.at[g]], rows_v.at[b], sem_g[b])

        def g_wait(g, b):
            pltpu.make_async_copy(
                table_hbm.at[idx_v.at[g]], rows_v.at[b], sem_g[b]
            ).wait()

        def o_start(g, b):
            pltpu.async_copy(
                rows_v.at[b], out_hbm.at[pl.ds(seq0, seqs_per_w), g], sem_o[b]
            )

        def o_wait(g, b):
            pltpu.make_async_copy(
                rows_v.at[b], out_hbm.at[pl.ds(seq0, seqs_per_w), g], sem_o[b]
            ).wait()

        for g0 in range(ahead):
            g_start(g0, g0)

        def round_body(r, carry):
            for b in range(nbuf):
                g = r * nbuf + b
                bt = (b + ahead) % nbuf
                gp = g + ahead
                gv = gp - nbuf

                @pl.when(gp < n_chunks)
                def _():
                    @pl.when(gv >= 0)
                    def _():
                        o_wait(gv, bt)

                    g_start(gp, bt)

                g_wait(g, b)
                p_row = [pe_v[g, pl.ds(j * LANES, LANES)] for j in range(D // LANES)]

                def t_body(t, c):
                    for j in range(D // LANES):
                        s = pl.ds(j * LANES, LANES)
                        rows_v[b, t, s] = rows_v[b, t, s] * scale + p_row[j]
                    return c

                lax.fori_loop(0, seqs_per_w, t_body, 0, unroll=2)
                o_start(g, b)
            return carry

        lax.fori_loop(0, n_rounds, round_body, 0, unroll=False)

        for b in range(nbuf):
            o_wait(n_chunks - nbuf + b, b)

    return k(inputs_t, table, pe)


def kernel(inputs, embed_weight):
    batch, seq_len = inputs.shape
    return _embed_sc(inputs.T, embed_weight, batch, seq_len)
